# Initial kernel scaffold; baseline (speedup 1.0000x reference)
import jax
import jax.numpy as jnp
from jax.experimental import pallas as pl


def kernel(encoder_outputs, attention_weights):
    raise NotImplementedError("write your pallas kernel here")


# speedup vs baseline: 1.1360x; 1.1360x over previous
import functools

import jax
import jax.numpy as jnp
from jax.experimental import pallas as pl
from jax.experimental.pallas import tpu as pltpu


def _attn_pool_kernel(x_ref, w_ref, o_ref, *, sc: int):
    bb, s, h = x_ref.shape
    nc = s // sc
    w = w_ref[...][None, :, :]

    m = None
    for c in range(nc):
        xc = x_ref[:, c * sc:(c + 1) * sc, :]
        scores = jnp.sum(jnp.tanh(xc) * w, axis=-1, keepdims=True)
        cm = jnp.max(scores, axis=1, keepdims=True)
        m = cm if m is None else jnp.maximum(m, cm)

    d = jnp.zeros((bb, 1, 1), dtype=jnp.float32)
    acc = jnp.zeros((bb, 1, h), dtype=jnp.float32)
    for c in range(nc):
        xc = x_ref[:, c * sc:(c + 1) * sc, :]
        scores = jnp.sum(jnp.tanh(xc) * w, axis=-1, keepdims=True)
        p = jnp.exp(scores - m)
        d = d + jnp.sum(p, axis=1, keepdims=True)
        acc = acc + jnp.sum(xc * p, axis=1, keepdims=True)

    o_ref[...] = (acc / d).reshape(bb, h)


def kernel(encoder_outputs, attention_weights):
    b, s, h = encoder_outputs.shape
    w2 = attention_weights.reshape(1, h)

    bb = 8 if b % 8 == 0 else 1
    sc = min(s, 128)
    assert s % sc == 0

    return pl.pallas_call(
        functools.partial(_attn_pool_kernel, sc=sc),
        out_shape=jax.ShapeDtypeStruct((b, h), jnp.float32),
        grid=(b // bb,),
        in_specs=[
            pl.BlockSpec((bb, s, h), lambda i: (i, 0, 0)),
            pl.BlockSpec((1, h), lambda i: (0, 0)),
        ],
        out_specs=pl.BlockSpec((bb, h), lambda i: (i, 0)),
        compiler_params=pltpu.CompilerParams(
            dimension_semantics=("parallel",),
            vmem_limit_bytes=56 * 1024 * 1024,
        ),
        name="additive_attention_pool",
    )(encoder_outputs, w2)


# speedup vs baseline: 1.8173x; 1.5997x over previous
import functools

import jax
import jax.numpy as jnp
from jax.experimental import pallas as pl
from jax.experimental.pallas import tpu as pltpu


def _attn_pool_kernel(x_ref, w_ref, o_ref, *, sc: int):
    bb, s, h = x_ref.shape
    nc = s // sc
    w = w_ref[...][None, :, :]

    m = d = acc = None
    for c in range(nc):
        xc = x_ref[:, c * sc:(c + 1) * sc, :]
        scores = jnp.sum(jnp.tanh(xc) * w, axis=-1, keepdims=True)
        cm = jnp.max(scores, axis=1, keepdims=True)
        m_new = cm if m is None else jnp.maximum(m, cm)
        p = jnp.exp2(scores - m_new)
        pd = jnp.sum(p, axis=1, keepdims=True)
        pacc = jnp.sum(xc * p, axis=1, keepdims=True)
        if m is None:
            d, acc = pd, pacc
        else:
            alpha = jnp.exp2(m - m_new)
            d = d * alpha + pd
            acc = acc * alpha + pacc
        m = m_new

    o_ref[...] = (acc / d).reshape(bb, h)


def kernel(encoder_outputs, attention_weights):
    b, s, h = encoder_outputs.shape
    w2 = (attention_weights * jnp.float32(1.4426950408889634)).reshape(1, h)

    bb = 8 if b % 8 == 0 else 1
    sc = min(s, 128)
    assert s % sc == 0

    return pl.pallas_call(
        functools.partial(_attn_pool_kernel, sc=sc),
        out_shape=jax.ShapeDtypeStruct((b, h), jnp.float32),
        grid=(b // bb,),
        in_specs=[
            pl.BlockSpec((bb, s, h), lambda i: (i, 0, 0)),
            pl.BlockSpec((1, h), lambda i: (0, 0)),
        ],
        out_specs=pl.BlockSpec((bb, h), lambda i: (i, 0)),
        compiler_params=pltpu.CompilerParams(
            dimension_semantics=("parallel",),
            vmem_limit_bytes=56 * 1024 * 1024,
        ),
        name="additive_attention_pool",
    )(encoder_outputs, w2)
